# baseline (speedup 1.0000x reference)
import jax
import jax.numpy as jnp
from jax import lax
from jax.experimental import pallas as pl
from jax.experimental.pallas import tpu as pltpu
from jax.experimental.pallas import tpu_sc as plsc

NC, NS, L = 2, 16, 16
NW = NC * NS
ROWS, COLS = 16384, 200
N_TOK = ROWS * COLS
JB, IB = COLS // 8, ROWS // 128
IBB = 4
UNITS = JB * (IB // IBB)
UPW = UNITS // NW
U_TOK = IBB * 1024
U_OUT = 2 * U_TOK

_PIO2_HI = 1.5707855224609375
_PIO2_LO = 1.0804334123550503e-05
_TWO_OVER_PI = 0.6366197723675814


def _sincos_table(scal_v, table_v):
    idx0 = jnp.zeros((L,), jnp.int32)
    a = plsc.load_gather(scal_v, [idx0 + 8])
    start = plsc.load_gather(scal_v, [idx0 + 24])
    stride = plsc.load_gather(scal_v, [idx0 + 40])
    d = lax.iota(jnp.int32, L).astype(jnp.float32)
    ang = start + d * stride
    kf = ang * _TWO_OVER_PI
    ki = jnp.where(kf >= 0, kf + 0.5, kf - 0.5).astype(jnp.int32)
    kx = ki.astype(jnp.float32)
    r = (ang - kx * _PIO2_HI) - kx * _PIO2_LO
    q = ki & 3
    r2 = r * r
    sp = r * (1.0 + r2 * (-1.0 / 6 + r2 * (1.0 / 120 + r2 * (-1.0 / 5040))))
    cp = 1.0 + r2 * (-1.0 / 2 + r2 * (1.0 / 24 + r2 * (
        -1.0 / 720 + r2 * (1.0 / 40320))))
    cos_v = jnp.where(q == 0, cp, jnp.where(q == 1, -sp,
                      jnp.where(q == 2, -cp, sp)))
    sin_v = jnp.where(q == 0, sp, jnp.where(q == 1, cp,
                      jnp.where(q == 2, -sp, -cp)))
    table_v[pl.ds(0, L)] = a * cos_v
    table_v[pl.ds(16, L)] = a * sin_v


def _sc_body(a_hbm, s_hbm, d_hbm, tok_hbm, out_hbm, scal_v, table_v,
             tok_v0, tok_v1, out_v0, out_v1, isem0, isem1, osem0, osem1):
    toks, outs = (tok_v0, tok_v1), (out_v0, out_v1)
    isems, osems = (isem0, isem1), (osem0, osem1)
    wid = lax.axis_index("s") * NC + lax.axis_index("c")
    base = wid * UPW
    pltpu.sync_copy(a_hbm, scal_v.at[pl.ds(8, 1)])
    pltpu.sync_copy(s_hbm, scal_v.at[pl.ds(24, 1)])
    pltpu.sync_copy(d_hbm, scal_v.at[pl.ds(40, 1)])
    _sincos_table(scal_v, table_v)

    def start_in(n, b):
        u = base + n
        jb = u >> 5
        ibb = u & 31
        pltpu.async_copy(
            tok_hbm.at[pl.ds(jb * 131072 + ibb * U_TOK, U_TOK)], toks[b],
            isems[b])

    def drain_in(b):
        pltpu.make_async_copy(
            tok_hbm.at[pl.ds(0, U_TOK)], toks[b], isems[b]).wait()

    def start_out(n, b):
        u = base + n
        jb = u >> 5
        ibb = u & 31
        pltpu.async_copy(
            outs[b],
            out_hbm.at[pl.ds(jb * 8, 8), pl.ds(ibb * IBB, IBB)],
            osems[b])

    def drain_out(b):
        pltpu.make_async_copy(
            out_hbm.at[pl.ds(0, 8), pl.ds(0, IBB)], outs[b], osems[b]).wait()

    def compute(b):
        tok_ref, out_ref = toks[b], outs[b]

        @plsc.parallel_loop(0, 256, unroll=8)
        def _vec(v):
            jr = v >> 5
            ibl = (v >> 3) & 3
            s = v & 7
            src = ibl * 1024 + jr * 128 + s * 16
            col = s * 16
            t = tok_ref[pl.ds(src, 16)]
            out_ref[jr, ibl, 0, pl.ds(col, 16)] = plsc.load_gather(
                table_v, [t])
            out_ref[jr, ibl, 1, pl.ds(col, 16)] = plsc.load_gather(
                table_v, [t + 16])

    start_in(0, 0)
    start_in(1, 1)

    @pl.loop(0, UPW - 1, step=2)
    def _g(g):
        for b in range(2):
            n = g + b
            drain_in(b)

            @pl.when(n >= 2)
            def _do(b=b):
                drain_out(b)

            compute(b)
            start_out(n, b)

            @pl.when(n + 2 < UPW)
            def _di(n=n, b=b):
                start_in(n + 2, b)

    drain_in(0)
    drain_out(0)
    compute(0)
    start_out(UPW - 1, 0)
    drain_out(1)
    drain_out(0)


def kernel(tokens, arc_A, arc_start, arc_stride):
    tok_flat = (
        tokens.T.reshape(JB, 8, IB, 128).transpose(0, 2, 1, 3).reshape(-1))
    sc = pl.kernel(
        _sc_body,
        out_type=jax.ShapeDtypeStruct((COLS, IB, 2, 128), jnp.float32),
        mesh=plsc.VectorSubcoreMesh(core_axis_name="c", subcore_axis_name="s"),
        compiler_params=pltpu.CompilerParams(needs_layout_passes=False),
        scratch_types=[
            pltpu.VMEM((48,), jnp.float32),
            pltpu.VMEM((32,), jnp.float32),
            pltpu.VMEM((U_TOK,), jnp.int32),
            pltpu.VMEM((U_TOK,), jnp.int32),
            pltpu.VMEM((8, IBB, 2, 128), jnp.float32),
            pltpu.VMEM((8, IBB, 2, 128), jnp.float32),
            pltpu.SemaphoreType.DMA,
            pltpu.SemaphoreType.DMA,
            pltpu.SemaphoreType.DMA,
            pltpu.SemaphoreType.DMA,
        ],
    )
    out = sc(arc_A.reshape(1), arc_start.reshape(1), arc_stride.reshape(1),
             tok_flat)
    return out.transpose(1, 3, 0, 2).reshape(ROWS, COLS, 2)


# speedup vs baseline: 1.0160x; 1.0160x over previous
import jax
import jax.numpy as jnp
from jax import lax
from jax.experimental import pallas as pl
from jax.experimental.pallas import tpu as pltpu
from jax.experimental.pallas import tpu_sc as plsc

NC, NS, L = 2, 16, 16
NW = NC * NS
ROWS, COLS = 16384, 200
N_TOK = ROWS * COLS
JB, IB = COLS // 8, ROWS // 128
IBB = 4
UNITS = JB * (IB // IBB)
UPW = UNITS // NW
U_TOK = IBB * 1024
U_OUT = 2 * U_TOK

_PIO2_HI = 1.5707855224609375
_PIO2_LO = 1.0804334123550503e-05
_TWO_OVER_PI = 0.6366197723675814


def _sincos_table(scal_v, table_v):
    idx0 = jnp.zeros((L,), jnp.int32)
    a = plsc.load_gather(scal_v, [idx0 + 8])
    start = plsc.load_gather(scal_v, [idx0 + 24])
    stride = plsc.load_gather(scal_v, [idx0 + 40])
    d = lax.iota(jnp.int32, L).astype(jnp.float32)
    ang = start + d * stride
    kf = ang * _TWO_OVER_PI
    ki = jnp.where(kf >= 0, kf + 0.5, kf - 0.5).astype(jnp.int32)
    kx = ki.astype(jnp.float32)
    r = (ang - kx * _PIO2_HI) - kx * _PIO2_LO
    q = ki & 3
    r2 = r * r
    sp = r * (1.0 + r2 * (-1.0 / 6 + r2 * (1.0 / 120 + r2 * (-1.0 / 5040))))
    cp = 1.0 + r2 * (-1.0 / 2 + r2 * (1.0 / 24 + r2 * (
        -1.0 / 720 + r2 * (1.0 / 40320))))
    cos_v = jnp.where(q == 0, cp, jnp.where(q == 1, -sp,
                      jnp.where(q == 2, -cp, sp)))
    sin_v = jnp.where(q == 0, sp, jnp.where(q == 1, cp,
                      jnp.where(q == 2, -sp, -cp)))
    table_v[pl.ds(0, L)] = a * cos_v
    table_v[pl.ds(16, L)] = a * sin_v


def _sc_body(a_hbm, s_hbm, d_hbm, tok_hbm, out_hbm, scal_v, table_v,
             tok_v0, tok_v1, out_v0, out_v1, isem0, isem1, osem0, osem1):
    toks, outs = (tok_v0, tok_v1), (out_v0, out_v1)
    isems, osems = (isem0, isem1), (osem0, osem1)
    wid = lax.axis_index("s") * NC + lax.axis_index("c")
    base = wid * UPW

    def start_in(n, b):
        u = base + n
        jb = u >> 5
        ibb = u & 31
        pltpu.async_copy(
            tok_hbm.at[pl.ds(jb * 131072 + ibb * U_TOK, U_TOK)], toks[b],
            isems[b])

    def drain_in(b):
        pltpu.make_async_copy(
            tok_hbm.at[pl.ds(0, U_TOK)], toks[b], isems[b]).wait()

    def start_out(n, b):
        u = base + n
        jb = u >> 5
        ibb = u & 31
        pltpu.async_copy(
            outs[b],
            out_hbm.at[pl.ds(jb * 8, 8), pl.ds(ibb * IBB, IBB)],
            osems[b])

    def drain_out(b):
        pltpu.make_async_copy(
            out_hbm.at[pl.ds(0, 8), pl.ds(0, IBB)], outs[b], osems[b]).wait()

    def compute(b):
        tok_ref, out_ref = toks[b], outs[b]

        @plsc.parallel_loop(0, 256, unroll=8)
        def _vec(v):
            jr = v >> 5
            ibl = (v >> 3) & 3
            s = v & 7
            src = ibl * 1024 + jr * 128 + s * 16
            col = s * 16
            t = tok_ref[pl.ds(src, 16)]
            out_ref[jr, ibl, 0, pl.ds(col, 16)] = plsc.load_gather(
                table_v, [t])
            out_ref[jr, ibl, 1, pl.ds(col, 16)] = plsc.load_gather(
                table_v, [t + 16])

    start_in(0, 0)
    start_in(1, 1)
    pltpu.sync_copy(a_hbm, scal_v.at[pl.ds(8, 1)])
    pltpu.sync_copy(s_hbm, scal_v.at[pl.ds(24, 1)])
    pltpu.sync_copy(d_hbm, scal_v.at[pl.ds(40, 1)])
    _sincos_table(scal_v, table_v)

    @pl.loop(0, UPW - 1, step=2)
    def _g(g):
        for b in range(2):
            n = g + b
            drain_in(b)

            @pl.when(n >= 2)
            def _do(b=b):
                drain_out(b)

            compute(b)
            start_out(n, b)

            @pl.when(n + 2 < UPW)
            def _di(n=n, b=b):
                start_in(n + 2, b)

    drain_in(0)
    drain_out(0)
    compute(0)
    start_out(UPW - 1, 0)
    drain_out(1)
    drain_out(0)


def kernel(tokens, arc_A, arc_start, arc_stride):
    tok_flat = (
        tokens.T.reshape(JB, 8, IB, 128).transpose(0, 2, 1, 3).reshape(-1))
    sc = pl.kernel(
        _sc_body,
        out_type=jax.ShapeDtypeStruct((COLS, IB, 2, 128), jnp.float32),
        mesh=plsc.VectorSubcoreMesh(core_axis_name="c", subcore_axis_name="s"),
        compiler_params=pltpu.CompilerParams(needs_layout_passes=False),
        scratch_types=[
            pltpu.VMEM((48,), jnp.float32),
            pltpu.VMEM((32,), jnp.float32),
            pltpu.VMEM((U_TOK,), jnp.int32),
            pltpu.VMEM((U_TOK,), jnp.int32),
            pltpu.VMEM((8, IBB, 2, 128), jnp.float32),
            pltpu.VMEM((8, IBB, 2, 128), jnp.float32),
            pltpu.SemaphoreType.DMA,
            pltpu.SemaphoreType.DMA,
            pltpu.SemaphoreType.DMA,
            pltpu.SemaphoreType.DMA,
        ],
    )
    out = sc(arc_A.reshape(1), arc_start.reshape(1), arc_stride.reshape(1),
             tok_flat)
    return out.transpose(1, 3, 0, 2).reshape(ROWS, COLS, 2)
